# Initial kernel scaffold; baseline (speedup 1.0000x reference)
import jax
import jax.numpy as jnp
from jax.experimental import pallas as pl


def kernel(x, mesh_pos, grid_pos, mesh_to_grid_edges, ip_w1, ip_b1, ip_w2, ip_b2, ip_w3, ip_b3, msg_w1, msg_b1, msg_w2, msg_b2, msg_w3, msg_b3):
    raise NotImplementedError("write your pallas kernel here")


# speedup vs baseline: 2.6880x; 2.6880x over previous
import functools

import jax
import jax.numpy as jnp
import numpy as np
from jax import lax
from jax.experimental import pallas as pl
from jax.experimental.pallas import tpu as pltpu
from jax.experimental.pallas import tpu_sc as plsc

INPUT_DIM = 16
HIDDEN = 128
NUM_GRID = 32768
N_MESH = 10000
N_EDGES = 320000
PE_DIM = 64
XMW = 256
GPW = 128

NC = 2
NS = 16
NW = NC * NS

CG = 256
G_NSUB = N_EDGES // CG
G_KMAX = (G_NSUB + NW - 1) // NW

GCH = 2048
NCHUNK = NUM_GRID // GCH
CS = 256
CS_LOG = 8
S_KMAX = (N_EDGES // CS + NS - 1) // NS
ROWS_PT = GCH // NS


def _sincos_consts():
    dim, ndim = PE_DIM, 3
    ndim_padding = dim % ndim
    dim_per_ndim = (dim - ndim_padding) // ndim
    sincos_padding = dim_per_ndim % 2
    padding = ndim_padding + sincos_padding * ndim
    eff = (dim - padding) // ndim
    arange = np.arange(0, eff, 2, dtype=np.float32)
    omega = 1.0 / (10000.0 ** (arange / eff))
    n_om = omega.shape[0]
    wpre = np.zeros((ndim, dim), np.float32)
    msin = np.zeros((1, dim), np.float32)
    mcos = np.zeros((1, dim), np.float32)
    for d in range(ndim):
        base = d * (2 * n_om)
        wpre[d, base:base + n_om] = omega
        wpre[d, base + n_om:base + 2 * n_om] = omega
        msin[0, base:base + n_om] = 1.0
        mcos[0, base + n_om:base + 2 * n_om] = 1.0
    return wpre, msin, mcos


def _erf(v):
    p = 0.3275911
    a1, a2, a3, a4, a5 = (0.254829592, -0.284496736, 1.421413741,
                          -1.453152027, 1.061405429)
    s = jnp.sign(v)
    av = jnp.abs(v)
    t = 1.0 / (1.0 + p * av)
    poly = t * (a1 + t * (a2 + t * (a3 + t * (a4 + t * a5))))
    return s * (1.0 - poly * jnp.exp(-av * av))


def _gelu(v):
    return 0.5 * v * (1.0 + _erf(v * 0.7071067811865475))


def _xm_body(x_ref, pos_ref, w1, b1, w2, b2, w3, b3, wpre, msin, mcos, out_ref):
    h = _gelu(jnp.dot(x_ref[...], w1[...], preferred_element_type=jnp.float32)
              + b1[...])
    h = _gelu(jnp.dot(h, w2[...], preferred_element_type=jnp.float32) + b2[...])
    h = jnp.dot(h, w3[...], preferred_element_type=jnp.float32) + b3[...]
    pos = pos_ref[...]
    pre = (pos[:, 0:1] * wpre[0:1, :] + pos[:, 1:2] * wpre[1:2, :]
           + pos[:, 2:3] * wpre[2:3, :])
    emb = jnp.sin(pre) * msin[...] + jnp.cos(pre) * mcos[...]
    pad = jnp.zeros((h.shape[0], XMW - HIDDEN - PE_DIM), jnp.float32)
    out_ref[...] = jnp.concatenate([h, emb, pad], axis=1)


def _make_xm(x, mesh_pos, w1, b1, w2, b2, w3, b3, wpre, msin, mcos):
    rb = 1000
    grid = N_MESH // rb
    return pl.pallas_call(
        _xm_body,
        grid=(grid,),
        in_specs=[
            pl.BlockSpec((rb, INPUT_DIM), lambda i: (i, 0)),
            pl.BlockSpec((rb, 3), lambda i: (i, 0)),
            pl.BlockSpec((INPUT_DIM, HIDDEN), lambda i: (0, 0)),
            pl.BlockSpec((1, HIDDEN), lambda i: (0, 0)),
            pl.BlockSpec((HIDDEN, HIDDEN), lambda i: (0, 0)),
            pl.BlockSpec((1, HIDDEN), lambda i: (0, 0)),
            pl.BlockSpec((HIDDEN, HIDDEN), lambda i: (0, 0)),
            pl.BlockSpec((1, HIDDEN), lambda i: (0, 0)),
            pl.BlockSpec((3, PE_DIM), lambda i: (0, 0)),
            pl.BlockSpec((1, PE_DIM), lambda i: (0, 0)),
            pl.BlockSpec((1, PE_DIM), lambda i: (0, 0)),
        ],
        out_specs=pl.BlockSpec((rb, XMW), lambda i: (i, 0)),
        out_shape=jax.ShapeDtypeStruct((N_MESH, XMW), jnp.float32),
    )(x, mesh_pos, w1, b1, w2, b2, w3, b3, wpre, msin, mcos)


def _gp_body(pos_ref, wpre, msin, mcos, out_ref):
    pos = pos_ref[...]
    pre = (pos[:, 0:1] * wpre[0:1, :] + pos[:, 1:2] * wpre[1:2, :]
           + pos[:, 2:3] * wpre[2:3, :])
    emb = jnp.sin(pre) * msin[...] + jnp.cos(pre) * mcos[...]
    pad = jnp.zeros((pos.shape[0], GPW - PE_DIM), jnp.float32)
    out_ref[...] = jnp.concatenate([emb, pad], axis=1)


def _make_gp(grid_pos, wpre, msin, mcos):
    rb = 2048
    grid = NUM_GRID // rb
    return pl.pallas_call(
        _gp_body,
        grid=(grid,),
        in_specs=[
            pl.BlockSpec((rb, 3), lambda i: (i, 0)),
            pl.BlockSpec((3, PE_DIM), lambda i: (0, 0)),
            pl.BlockSpec((1, PE_DIM), lambda i: (0, 0)),
            pl.BlockSpec((1, PE_DIM), lambda i: (0, 0)),
        ],
        out_specs=pl.BlockSpec((rb, GPW), lambda i: (i, 0)),
        out_shape=jax.ShapeDtypeStruct((NUM_GRID, GPW), jnp.float32),
    )(grid_pos, wpre, msin, mcos)


def _bnd_body(gi_ref, out_ref):
    v = gi_ref[...]
    acc = jnp.zeros((1, 128), jnp.int32)
    for c in range(1, NCHUNK):
        cnt = jnp.sum((v < c * GCH).astype(jnp.int32))
        onehot = (lax.broadcasted_iota(jnp.int32, (1, 128), 1) == c)
        acc = acc + jnp.where(onehot, cnt, 0)
    out_ref[...] = acc


def _make_bounds(gi2d):
    out = pl.pallas_call(
        _bnd_body,
        in_specs=[pl.BlockSpec((N_EDGES // 128, 128), lambda: (0, 0))],
        out_specs=pl.BlockSpec((1, 128), lambda: (0, 0)),
        out_shape=jax.ShapeDtypeStruct((1, 128), jnp.int32),
    )(gi2d)
    bnd = jnp.zeros((32,), jnp.int32)
    bnd = bnd.at[1:NCHUNK].set(out[0, 1:NCHUNK])
    bnd = bnd.at[NCHUNK].set(N_EDGES)
    return bnd


def _gather_body(xm_hbm, gp_hbm, mi_hbm, gi_hbm, mxm_out, mgp_out,
                 idxm, idxg, xmr, gpr, semx, semg):
    wid = lax.axis_index("s") * NC + lax.axis_index("c")

    def body(kk, carry):
        s = wid + kk * NW

        @pl.when(s < G_NSUB)
        def _():
            e0 = pl.multiple_of(s * CG, CG)
            pltpu.sync_copy(mi_hbm.at[pl.ds(e0, CG)], idxm)
            pltpu.sync_copy(gi_hbm.at[pl.ds(e0, CG)], idxg)
            cps = []
            for j in range(CG // 128):
                cps.append(pltpu.async_copy(
                    xm_hbm.at[idxm.at[pl.ds(j * 128, 128)]],
                    xmr.at[pl.ds(j * 128, 128), :], semx))
                cps.append(pltpu.async_copy(
                    gp_hbm.at[idxg.at[pl.ds(j * 128, 128)]],
                    gpr.at[pl.ds(j * 128, 128), :], semg))
            for cp in cps:
                cp.wait()
            pltpu.sync_copy(xmr, mxm_out.at[pl.ds(e0, CG), :])
            pltpu.sync_copy(gpr, mgp_out.at[pl.ds(e0, CG), :])

        return carry

    lax.fori_loop(0, G_KMAX, body, 0)


def _run_gather(xm, gp, mi, gi):
    mesh = plsc.VectorSubcoreMesh(core_axis_name="c", subcore_axis_name="s")
    f = functools.partial(
        pl.kernel,
        out_type=[
            jax.ShapeDtypeStruct((N_EDGES, XMW), jnp.float32),
            jax.ShapeDtypeStruct((N_EDGES, GPW), jnp.float32),
        ],
        mesh=mesh,
        scratch_types=[
            pltpu.VMEM((CG,), jnp.int32),
            pltpu.VMEM((CG,), jnp.int32),
            pltpu.VMEM((CG, XMW), jnp.float32),
            pltpu.VMEM((CG, GPW), jnp.float32),
            pltpu.SemaphoreType.DMA,
            pltpu.SemaphoreType.DMA,
        ],
    )(_gather_body)
    return f(xm, gp, mi, gi)


def _mlp_body(mx_ref, mg_ref, w1a, w1b, b1, w2, b2, w3, b3, out_ref):
    e = (jnp.dot(mx_ref[...], w1a[...], preferred_element_type=jnp.float32)
         + jnp.dot(mg_ref[...], w1b[...], preferred_element_type=jnp.float32)
         + b1[...])
    e = _gelu(e)
    e = _gelu(jnp.dot(e, w2[...], preferred_element_type=jnp.float32) + b2[...])
    out_ref[...] = (jnp.dot(e, w3[...], preferred_element_type=jnp.float32)
                    + b3[...])


def _run_mlp(mxm, mgp, w1a, w1b, b1, w2, b2, w3, b3):
    rb = 512
    grid = N_EDGES // rb
    d2 = 2 * HIDDEN
    return pl.pallas_call(
        _mlp_body,
        grid=(grid,),
        in_specs=[
            pl.BlockSpec((rb, XMW), lambda i: (i, 0)),
            pl.BlockSpec((rb, GPW), lambda i: (i, 0)),
            pl.BlockSpec((XMW, d2), lambda i: (0, 0)),
            pl.BlockSpec((GPW, d2), lambda i: (0, 0)),
            pl.BlockSpec((1, d2), lambda i: (0, 0)),
            pl.BlockSpec((d2, d2), lambda i: (0, 0)),
            pl.BlockSpec((1, d2), lambda i: (0, 0)),
            pl.BlockSpec((d2, HIDDEN), lambda i: (0, 0)),
            pl.BlockSpec((1, HIDDEN), lambda i: (0, 0)),
        ],
        out_specs=pl.BlockSpec((rb, HIDDEN), lambda i: (i, 0)),
        out_shape=jax.ShapeDtypeStruct((N_EDGES, HIDDEN), jnp.float32),
    )(mxm, mgp, w1a, w1b, b1, w2, b2, w3, b3)


def _scatter_body(m_hbm, gi_hbm, bnd_hbm, sums_out, cnts_out,
                  idxr, idx2, mrows, ones_t, zb, bndv, acc, cacc):
    cid = lax.axis_index("c")
    sid = lax.axis_index("s")
    zero16 = jnp.zeros((16,), jnp.float32)
    one16 = jnp.ones((16,), jnp.float32)
    iota16 = lax.iota(jnp.int32, 16)

    def init_row(r, carry):
        for cv in range(8):
            zb[r, pl.ds(cv * 16, 16)] = zero16
            ones_t[r, pl.ds(cv * 16, 16)] = one16
        return carry

    lax.fori_loop(0, 128, init_row, 0)

    pltpu.sync_copy(bnd_hbm, bndv)
    base = sid * ROWS_PT

    def chunk_body(k, chunk_carry):
        c = cid * (NCHUNK // NC) + k
        g0 = c * GCH
        bwin = bndv[pl.ds(c, 16)]
        lo = bwin[0]
        hi = bwin[1]
        a_lo = lo & -8
        a_hi = (hi + 7) & -8
        nsub = (a_hi - a_lo + (CS - 1)) >> CS_LOG

        def zero_acc(j, carry):
            pltpu.sync_copy(zb, acc.at[pl.ds(base + j * 128, 128), :])
            pltpu.sync_copy(zb, cacc.at[pl.ds(base + j * 128, 128), :])
            return carry

        lax.fori_loop(0, ROWS_PT // 128, zero_acc, 0)

        @pl.when(sid == NS - 1)
        def _():
            pltpu.sync_copy(zb.at[pl.ds(0, 8), :], acc.at[pl.ds(GCH, 8), :])
            pltpu.sync_copy(zb.at[pl.ds(0, 8), :], cacc.at[pl.ds(GCH, 8), :])

        plsc.subcore_barrier()

        def sbody(i, carry):
            s_local = sid + i * NS

            @pl.when(s_local < nsub)
            def _():
                e0 = a_lo + s_local * CS
                e0r = pl.multiple_of(jnp.minimum(e0, N_EDGES - CS), 8)
                pltpu.sync_copy(gi_hbm.at[pl.ds(e0r, CS)], idxr)
                pltpu.sync_copy(m_hbm.at[pl.ds(e0r, CS), :], mrows)
                for j2 in range(CS // 16):
                    v = idxr[pl.ds(j2 * 16, 16)]
                    pos = e0r + j2 * 16 + iota16
                    vl = v - g0
                    ok = (pos >= e0) & (vl >= 0) & (vl < GCH)
                    vlc = jnp.where(ok, vl, GCH)
                    idx2[j2 // 8, pl.ds((j2 % 8) * 16, 16)] = vlc
                for j in range(CS // 128):
                    pltpu.sync_copy(mrows.at[pl.ds(j * 128, 128), :],
                                    acc.at[idx2.at[j]], add=True)
                    pltpu.sync_copy(ones_t, cacc.at[idx2.at[j]], add=True)

            return carry

        lax.fori_loop(0, S_KMAX, sbody, 0)
        plsc.subcore_barrier()

        ob = pl.multiple_of(g0 + base, 8)
        pltpu.sync_copy(acc.at[pl.ds(base, ROWS_PT), :],
                        sums_out.at[pl.ds(ob, ROWS_PT), :])
        pltpu.sync_copy(cacc.at[pl.ds(base, ROWS_PT), :],
                        cnts_out.at[pl.ds(ob, ROWS_PT), :])
        plsc.subcore_barrier()
        return chunk_carry

    lax.fori_loop(0, NCHUNK // NC, chunk_body, 0)


def _run_scatter(m, gidx, bnd):
    mesh = plsc.VectorSubcoreMesh(core_axis_name="c", subcore_axis_name="s")
    f = functools.partial(
        pl.kernel,
        out_type=[
            jax.ShapeDtypeStruct((NUM_GRID, HIDDEN), jnp.float32),
            jax.ShapeDtypeStruct((NUM_GRID, 128), jnp.float32),
        ],
        mesh=mesh,
        scratch_types=[
            pltpu.VMEM((CS,), jnp.int32),
            pltpu.VMEM((CS // 128, 128), jnp.int32),
            pltpu.VMEM((CS, HIDDEN), jnp.float32),
            pltpu.VMEM((128, 128), jnp.float32),
            pltpu.VMEM((128, 128), jnp.float32),
            pltpu.VMEM((32,), jnp.int32),
            pltpu.VMEM_SHARED((GCH + 8, HIDDEN), jnp.float32),
            pltpu.VMEM_SHARED((GCH + 8, 128), jnp.float32),
        ],
    )(_scatter_body)
    return f(m, gidx, bnd)


def _div_body(s_ref, c_ref, out_ref):
    cnt = jnp.maximum(c_ref[:, 0:1], 1.0)
    out_ref[...] = s_ref[...] / cnt


def _run_div(sums, cnts):
    rb = 2048
    grid = NUM_GRID // rb
    return pl.pallas_call(
        _div_body,
        grid=(grid,),
        in_specs=[
            pl.BlockSpec((rb, HIDDEN), lambda i: (i, 0)),
            pl.BlockSpec((rb, 128), lambda i: (i, 0)),
        ],
        out_specs=pl.BlockSpec((rb, HIDDEN), lambda i: (i, 0)),
        out_shape=jax.ShapeDtypeStruct((NUM_GRID, HIDDEN), jnp.float32),
    )(sums, cnts)


def kernel(x, mesh_pos, grid_pos, mesh_to_grid_edges, ip_w1, ip_b1, ip_w2,
           ip_b2, ip_w3, ip_b3, msg_w1, msg_b1, msg_w2, msg_b2, msg_w3,
           msg_b3):
    wpre_np, msin_np, mcos_np = _sincos_consts()
    wpre = jnp.asarray(wpre_np)
    msin = jnp.asarray(msin_np)
    mcos = jnp.asarray(mcos_np)

    grid_idx = mesh_to_grid_edges[:, 0]
    mesh_idx = mesh_to_grid_edges[:, 1]
    gi2d = grid_idx.reshape(N_EDGES // 128, 128)

    xm = _make_xm(x, mesh_pos, ip_w1, ip_b1.reshape(1, -1), ip_w2,
                  ip_b2.reshape(1, -1), ip_w3, ip_b3.reshape(1, -1),
                  wpre, msin, mcos)
    gp = _make_gp(grid_pos, wpre, msin, mcos)
    bnd = _make_bounds(gi2d)

    mxm, mgp = _run_gather(xm, gp, mesh_idx, grid_idx)

    d1 = HIDDEN + PE_DIM
    w1a = jnp.pad(msg_w1[:d1], ((0, XMW - d1), (0, 0)))
    w1b = jnp.pad(msg_w1[d1:], ((0, GPW - PE_DIM), (0, 0)))
    m = _run_mlp(mxm, mgp, w1a, w1b, msg_b1.reshape(1, -1),
                 msg_w2, msg_b2.reshape(1, -1), msg_w3,
                 msg_b3.reshape(1, -1))

    sums, cnts = _run_scatter(m, grid_idx, bnd)
    out = _run_div(sums, cnts)
    return out.reshape(1, NUM_GRID, HIDDEN)
